# baseline (speedup 1.0000x reference)
import jax
import jax.numpy as jnp
from jax.experimental import pallas as pl

_N = 524288
_ROWS = _N * 48 // 1024
_BR = 1024
_PAT = 7 * 48


def _copy_body(x_ref, o_ref):
    i = pl.program_id(0)
    o_ref[...] = x_ref[...]

    @pl.when(i == 0)
    def _patch():
        col = jax.lax.broadcasted_iota(jnp.int32, (1, 1024), 1)
        pat = (col % 48).astype(jnp.float32)
        o_ref[0:1, :] = jnp.where(col < _PAT, pat, x_ref[0:1, :])


def kernel(x):
    xv = x.reshape(_ROWS, 1024)
    out = pl.pallas_call(
        _copy_body,
        grid=(_ROWS // _BR,),
        in_specs=[pl.BlockSpec((_BR, 1024), lambda i: (i, 0))],
        out_specs=pl.BlockSpec((_BR, 1024), lambda i: (i, 0)),
        out_shape=jax.ShapeDtypeStruct((_ROWS, 1024), jnp.float32),
    )(xv)
    return out.reshape(_N, 6, 8)
